# baseline (speedup 1.0000x reference)
import functools

import jax
import jax.numpy as jnp
from jax import lax
from jax.experimental import pallas as pl
from jax.experimental.pallas import tpu as pltpu
from jax.experimental.pallas import tpu_sc as plsc

BATCH = 128
T_LEN = 32768
NCH = 4
OUT_LEN = 8192
MAX_T = T_LEN - OUT_LEN - 1

LANES = 128
ROWS = T_LEN * NCH // LANES
BPB = 8

FLAT_PER_BATCH = T_LEN * NCH
OUT_PER_BATCH = OUT_LEN * NCH
WIN_ELEMS = OUT_PER_BATCH + 8


def _bf16_rtne(x):
    u = lax.bitcast_convert_type(x, jnp.uint32)
    r = u + jnp.uint32(0x7FFF) + ((u >> jnp.uint32(16)) & jnp.uint32(1))
    return lax.bitcast_convert_type(r & jnp.uint32(0xFFFF0000), jnp.float32)


def _starts_body(x_ref, w_ref, b_ref, o_ref):
    rowsum = jnp.sum(x_ref[...], axis=1)
    t = rowsum
    for sh in (64, 32, 16, 8, 4):
        t = t + pltpu.roll(t, LANES - sh, axis=1)
    m = t * jnp.float32(1.0 / T_LEN)
    mb = _bf16_rtne(m)
    wb = _bf16_rtne(w_ref[...])
    y = mb[:, 0:1] * wb[0:1, :]
    for c in range(1, NCH):
        y = y + mb[:, c:c + 1] * wb[c:c + 1, :]
    y = jax.nn.sigmoid(y + b_ref[...])
    st = jnp.floor(y * jnp.float32(T_LEN - 1))
    st = jnp.clip(st, 0.0, jnp.float32(MAX_T))
    o_ref[...] = st.astype(jnp.int32)


def _compute_starts(x3, wp, brow):
    return pl.pallas_call(
        _starts_body,
        grid=(BATCH // BPB,),
        in_specs=[
            pl.BlockSpec((BPB, ROWS, LANES), lambda i: (i, 0, 0)),
            pl.BlockSpec((BPB, LANES), lambda i: (0, 0)),
            pl.BlockSpec((BPB, LANES), lambda i: (0, 0)),
        ],
        out_specs=pl.BlockSpec((BPB, LANES), lambda i: (i, 0)),
        out_shape=jax.ShapeDtypeStruct((BATCH, LANES), jnp.int32),
    )(x3, wp, brow)


def _iota16():
    return lax.iota(jnp.int32, 16)


def _gather_body(x_hbm, st_hbm, out_hbm, st_v, win0, win1, out_v,
                 sem0, sem1):
    nc = 2
    wid = lax.axis_index("s") * nc + lax.axis_index("c")
    pltpu.sync_copy(st_hbm.at[pl.ds(wid * 16, 16)], st_v)
    sv = st_v[...]
    iota = _iota16()
    four_iota = iota * 4

    wins = (win0, win1)
    sems = (sem0, sem1)

    def start_dma(i):
        k, c = divmod(i, NCH)
        b = 4 * wid + k
        s = sv[4 * k + c]
        se = s - (s & 1)
        base = pl.multiple_of(b * FLAT_PER_BATCH + 4 * se, 8)
        d = 4 * (s - se) + c
        buf = i % 2
        cp = pltpu.async_copy(
            x_hbm.at[pl.ds(base, WIN_ELEMS)], wins[buf], sems[buf])
        return cp, d, c, b

    def extract(buf_ref, d, c):
        src0 = four_iota + d
        dst0 = four_iota + c

        def body(j, carry):
            off = j * 64
            v = plsc.load_gather(buf_ref, [src0 + off])
            plsc.store_scatter(out_v, [dst0 + off], v)
            return carry

        def outer(jj, carry):
            for u in range(16):
                body(jj * 16 + u, 0)
            return carry

        lax.fori_loop(0, 32, outer, 0)

    cur = start_dma(0)
    for i in range(16):
        nxt = start_dma(i + 1) if i + 1 < 16 else None
        cp, d, c, b = cur
        cp.wait()
        extract(wins[i % 2], d, c)
        if c == NCH - 1:
            pltpu.sync_copy(out_v, out_hbm.at[pl.ds(b * OUT_PER_BATCH,
                                                    OUT_PER_BATCH)])
        cur = nxt


def _gather_sc(x_flat, st4):
    mesh = plsc.VectorSubcoreMesh(core_axis_name="c", subcore_axis_name="s")
    k = functools.partial(
        pl.kernel,
        mesh=mesh,
        out_type=jax.ShapeDtypeStruct((BATCH * OUT_PER_BATCH,), jnp.float32),
        scratch_types=[
            pltpu.VMEM((16,), jnp.int32),
            pltpu.VMEM((WIN_ELEMS,), jnp.float32),
            pltpu.VMEM((WIN_ELEMS,), jnp.float32),
            pltpu.VMEM((OUT_PER_BATCH,), jnp.float32),
            pltpu.SemaphoreType.DMA,
            pltpu.SemaphoreType.DMA,
        ],
        compiler_params=pltpu.CompilerParams(needs_layout_passes=False),
    )(_gather_body)
    return k(x_flat, st4)


def kernel(X, W, b):
    batch, t_len, nch = X.shape
    assert (batch, t_len, nch) == (BATCH, T_LEN, NCH)
    x3 = X.reshape(BATCH, ROWS, LANES)
    wp = jnp.pad(W, ((0, BPB - NCH), (0, LANES - NCH)))
    brow = jnp.broadcast_to(
        jnp.pad(b, (0, LANES - NCH)).reshape(1, LANES), (BPB, LANES))
    starts = _compute_starts(x3, wp, brow)
    st4 = starts[:, :NCH].reshape(-1)
    out_flat = _gather_sc(X.reshape(-1), st4)
    return out_flat.reshape(BATCH, OUT_LEN, NCH)
